# baseline (speedup 1.0000x reference)
import functools

import jax
import jax.numpy as jnp
from jax import lax
from jax.experimental import pallas as pl
from jax.experimental.pallas import tpu as pltpu
from jax.experimental.pallas import tpu_sc as plsc

D = 64
NB = 4096
T = 200
BW = 128

_info = plsc.get_sparse_core_info()
NC, NS = _info.num_cores, _info.num_subcores
NW = NC * NS

_mesh = plsc.VectorSubcoreMesh(core_axis_name="c", subcore_axis_name="s")


@functools.partial(
    pl.kernel,
    mesh=_mesh,
    compiler_params=pltpu.CompilerParams(
        use_tc_tiling_on_sc=False, needs_layout_passes=False),
    out_type=jax.ShapeDtypeStruct((T, D // 8, NW, 8, BW), jnp.float32),
    scratch_types=[
        pltpu.VMEM((T, BW), jnp.int32),
        pltpu.VMEM((2, BW, D), jnp.float32),
        pltpu.VMEM((2, D // 8, 8, BW), jnp.float32),
        pltpu.SemaphoreType.DMA((2,)),
        pltpu.SemaphoreType.DMA((2,)),
    ],
)
def _emb_lookup(w_hbm, xt_hbm, out_hbm, idx_v, raw_v, trans_v, gsem, wsem):
    wid = lax.axis_index("s") * NC + lax.axis_index("c")
    b0 = wid * BW
    pltpu.sync_copy(xt_hbm.at[:, pl.ds(b0, BW)], idx_v)

    def g_copy(t, j):
        return pltpu.make_async_copy(
            w_hbm.at[idx_v.at[t]], raw_v.at[j], gsem.at[j])

    def wb_copy(t, j):
        return pltpu.make_async_copy(
            trans_v.at[j], out_hbm.at[t, :, wid], wsem.at[j])

    def transpose(j):
        def inner(i0, c):
            rows = lax.iota(jnp.int32, 16) + i0 * 16
            for d in range(D):
                cols = jnp.full((16,), d, jnp.int32)
                v = plsc.load_gather(raw_v.at[j], [rows, cols])
                trans_v.at[j][d // 8, d % 8, pl.ds(i0 * 16, 16)] = v
            return c
        lax.fori_loop(0, BW // 16, inner, 0)

    g_copy(0, 0).start()
    g_copy(1, 1).start()

    def pair(p, carry):
        t0 = p * 2
        for j in (0, 1):
            t = t0 + j
            g_copy(t, j).wait()

            @pl.when(t >= 2)
            def _():
                wb_copy(t - 2, j).wait()

            transpose(j)
            wb_copy(t, j).start()

            @pl.when(t + 2 < T)
            def _():
                g_copy(t + 2, j).start()

        return carry

    lax.fori_loop(0, T // 2, pair, 0)
    wb_copy(T - 2, 0).wait()
    wb_copy(T - 1, 1).wait()


def kernel(X, W):
    x_t = jnp.transpose(X).astype(jnp.int32)
    out5 = _emb_lookup(W, x_t)
    return jnp.transpose(out5, (2, 4, 0, 1, 3)).reshape(NB, T, D)


# speedup vs baseline: 2.5946x; 2.5946x over previous
import functools

import jax
import jax.numpy as jnp
from jax import lax
from jax.experimental import pallas as pl
from jax.experimental.pallas import tpu as pltpu
from jax.experimental.pallas import tpu_sc as plsc

D = 64
NB = 4096
T = 200
BW = 128

_info = plsc.get_sparse_core_info()
NC, NS = _info.num_cores, _info.num_subcores
NW = NC * NS

_mesh = plsc.VectorSubcoreMesh(core_axis_name="c", subcore_axis_name="s")


@functools.partial(
    pl.kernel,
    mesh=_mesh,
    compiler_params=pltpu.CompilerParams(
        use_tc_tiling_on_sc=False, needs_layout_passes=False,
        disable_bounds_checks=True),
    out_type=jax.ShapeDtypeStruct((T, D // 8, NW, 8, BW), jnp.float32),
    scratch_types=[
        pltpu.VMEM((T, BW), jnp.int32),
        pltpu.VMEM((2, BW, D), jnp.float32),
        pltpu.VMEM((2, D, BW), jnp.float32),
        pltpu.SemaphoreType.DMA((2,)),
        pltpu.SemaphoreType.DMA((2,)),
    ],
)
def _emb_lookup(w_hbm, xt_hbm, out_hbm, idx_v, raw_v, trans_v, gsem, wsem):
    wid = lax.axis_index("s") * NC + lax.axis_index("c")
    b0 = wid * BW
    pltpu.sync_copy(xt_hbm.at[:, pl.ds(b0, BW)], idx_v)

    def g_copy(t, j):
        return pltpu.make_async_copy(
            w_hbm.at[idx_v.at[t]], raw_v.at[j], gsem.at[j])

    def wb_copies(t, j):
        return [
            pltpu.make_async_copy(
                trans_v.at[j, pl.ds(dt * 8, 8)],
                out_hbm.at[t, dt, wid],
                wsem.at[j])
            for dt in range(D // 8)
        ]

    iota16 = lax.iota(jnp.int32, 16)
    row_ids = [iota16 + i0 * 16 for i0 in range(BW // 16)]

    def transpose(j):
        def per_d(d, c):
            colv = (iota16 + d) & (D - 1)
            for i0 in range(BW // 16):
                v = plsc.load_gather(raw_v.at[j], [row_ids[i0], colv])
                plsc.store_scatter(trans_v.at[j], [colv, row_ids[i0]], v)
            return c
        lax.fori_loop(0, D, per_d, 0)

    g_copy(0, 0).start()
    g_copy(1, 1).start()

    def pair(p, carry):
        t0 = p * 2
        for j in (0, 1):
            t = t0 + j
            g_copy(t, j).wait()

            @pl.when(t >= 2)
            def _():
                for cp in wb_copies(t - 2, j):
                    cp.wait()

            transpose(j)
            for cp in wb_copies(t, j):
                cp.start()

            @pl.when(t + 2 < T)
            def _():
                g_copy(t + 2, j).start()

        return carry

    lax.fori_loop(0, T // 2, pair, 0)
    for cp in wb_copies(T - 2, 0):
        cp.wait()
    for cp in wb_copies(T - 1, 1):
        cp.wait()


def kernel(X, W):
    x_t = jnp.transpose(X).astype(jnp.int32)
    out5 = _emb_lookup(W, x_t)
    return jnp.transpose(out5, (2, 4, 0, 1, 3)).reshape(NB, T, D)


# speedup vs baseline: 2.5954x; 1.0003x over previous
import functools

import jax
import jax.numpy as jnp
from jax import lax
from jax.experimental import pallas as pl
from jax.experimental.pallas import tpu as pltpu
from jax.experimental.pallas import tpu_sc as plsc

D = 64
NB = 4096
T = 200
BW = 128

_info = plsc.get_sparse_core_info()
NC, NS = _info.num_cores, _info.num_subcores
NW = NC * NS

_mesh = plsc.VectorSubcoreMesh(core_axis_name="c", subcore_axis_name="s")


@functools.partial(
    pl.kernel,
    mesh=_mesh,
    compiler_params=pltpu.CompilerParams(
        use_tc_tiling_on_sc=False, needs_layout_passes=False,
        disable_bounds_checks=True),
    out_type=jax.ShapeDtypeStruct((T, D // 8, NW, 8 * BW), jnp.float32),
    scratch_types=[
        pltpu.VMEM((T, BW), jnp.int32),
        pltpu.VMEM((2, BW, D), jnp.float32),
        pltpu.VMEM((2, D * BW), jnp.float32),
        pltpu.SemaphoreType.DMA((2,)),
        pltpu.SemaphoreType.DMA((2,)),
    ],
)
def _emb_lookup(w_hbm, xt_hbm, out_hbm, idx_v, raw_v, trans_v, gsem, wsem):
    wid = lax.axis_index("s") * NC + lax.axis_index("c")
    b0 = wid * BW
    pltpu.sync_copy(xt_hbm.at[:, pl.ds(b0, BW)], idx_v)

    def g_copy(t, j):
        return pltpu.make_async_copy(
            w_hbm.at[idx_v.at[t]], raw_v.at[j], gsem.at[j])

    def wb_copies(t, j):
        return [
            pltpu.make_async_copy(
                trans_v.at[j, pl.ds(dt * 8 * BW, 8 * BW)],
                out_hbm.at[t, dt, wid],
                wsem.at[j])
            for dt in range(D // 8)
        ]

    iota16 = lax.iota(jnp.int32, 16)
    row_ids = [iota16 + i0 * 16 for i0 in range(BW // 16)]

    def transpose(j):
        def per_d(d, c):
            colv = (iota16 + d) & (D - 1)
            colv128 = colv << 7
            for i0 in range(BW // 16):
                v = plsc.load_gather(raw_v.at[j], [row_ids[i0], colv])
                plsc.store_scatter(trans_v.at[j], [colv128 + row_ids[i0]], v)
            return c
        lax.fori_loop(0, D, per_d, 0)

    g_copy(0, 0).start()
    g_copy(1, 1).start()

    def pair(p, carry):
        t0 = p * 2
        for j in (0, 1):
            t = t0 + j
            g_copy(t, j).wait()

            @pl.when(t >= 2)
            def _():
                for cp in wb_copies(t - 2, j):
                    cp.wait()

            transpose(j)
            for cp in wb_copies(t, j):
                cp.start()

            @pl.when(t + 2 < T)
            def _():
                g_copy(t + 2, j).start()

        return carry

    lax.fori_loop(0, T // 2, pair, 0)
    for cp in wb_copies(T - 2, 0):
        cp.wait()
    for cp in wb_copies(T - 1, 1):
        cp.wait()


def kernel(X, W):
    x_t = jnp.transpose(X).astype(jnp.int32)
    out5 = _emb_lookup(W, x_t).reshape(T, D // 8, NW, 8, BW)
    return jnp.transpose(out5, (2, 4, 0, 1, 3)).reshape(NB, T, D)


# speedup vs baseline: 4.2729x; 1.6463x over previous
import functools

import jax
import jax.numpy as jnp
from jax import lax
from jax.experimental import pallas as pl
from jax.experimental.pallas import tpu as pltpu
from jax.experimental.pallas import tpu_sc as plsc

D = 64
NB = 4096
T = 200
BW = 128

_info = plsc.get_sparse_core_info()
NC, NS = _info.num_cores, _info.num_subcores
NW = NC * NS

_mesh = plsc.VectorSubcoreMesh(core_axis_name="c", subcore_axis_name="s")


@functools.partial(
    pl.kernel,
    mesh=_mesh,
    compiler_params=pltpu.CompilerParams(
        use_tc_tiling_on_sc=False, needs_layout_passes=False,
        disable_bounds_checks=True),
    out_type=jax.ShapeDtypeStruct((T, D // 8, NW, 8 * BW), jnp.float32),
    scratch_types=[
        pltpu.VMEM((T, BW), jnp.int32),
        pltpu.VMEM((2, BW, D), jnp.float32),
        pltpu.VMEM((2, D * BW), jnp.float32),
        pltpu.SemaphoreType.DMA((2,)),
        pltpu.SemaphoreType.DMA((2,)),
    ],
)
def _emb_lookup(w_hbm, xt_hbm, out_hbm, idx_v, raw_v, trans_v, gsem, wsem):
    wid = lax.axis_index("s") * NC + lax.axis_index("c")
    b0 = wid * BW
    pltpu.sync_copy(xt_hbm.at[:, pl.ds(b0, BW)], idx_v)

    def g_copy(t, j):
        return pltpu.make_async_copy(
            w_hbm.at[idx_v.at[t]], raw_v.at[j], gsem.at[j])

    def wb_copies(t, j):
        return [
            pltpu.make_async_copy(
                trans_v.at[j, pl.ds(dt * 8 * BW, 8 * BW)],
                out_hbm.at[t, dt, wid],
                wsem.at[j])
            for dt in range(D // 8)
        ]

    iota16 = lax.iota(jnp.int32, 16)
    row_ids = [iota16 + i0 * 16 for i0 in range(BW // 16)]

    def transpose(j):
        @plsc.parallel_loop(0, D, unroll=4)
        def per_d(d):
            colv = (iota16 + d) & (D - 1)
            colv128 = colv << 7
            for i0 in range(BW // 16):
                v = plsc.load_gather(raw_v.at[j], [row_ids[i0], colv])
                plsc.store_scatter(trans_v.at[j], [colv128 + row_ids[i0]], v)

    g_copy(0, 0).start()
    g_copy(1, 1).start()

    def pair(p, carry):
        t0 = p * 2
        for j in (0, 1):
            t = t0 + j
            g_copy(t, j).wait()

            @pl.when(t >= 2)
            def _():
                for cp in wb_copies(t - 2, j):
                    cp.wait()

            transpose(j)
            for cp in wb_copies(t, j):
                cp.start()

            @pl.when(t + 2 < T)
            def _():
                g_copy(t + 2, j).start()

        return carry

    lax.fori_loop(0, T // 2, pair, 0)
    for cp in wb_copies(T - 2, 0):
        cp.wait()
    for cp in wb_copies(T - 1, 1):
        cp.wait()


def kernel(X, W):
    x_t = jnp.transpose(X).astype(jnp.int32)
    out5 = _emb_lookup(W, x_t).reshape(T, D // 8, NW, 8, BW)
    return jnp.transpose(out5, (2, 4, 0, 1, 3)).reshape(NB, T, D)


# speedup vs baseline: 4.9868x; 1.1671x over previous
import functools

import jax
import jax.numpy as jnp
from jax import lax
from jax.experimental import pallas as pl
from jax.experimental.pallas import tpu as pltpu
from jax.experimental.pallas import tpu_sc as plsc

D = 64
NB = 4096
T = 200
BW = 128

_info = plsc.get_sparse_core_info()
NC, NS = _info.num_cores, _info.num_subcores
NW = NC * NS

_mesh = plsc.VectorSubcoreMesh(core_axis_name="c", subcore_axis_name="s")


@functools.partial(
    pl.kernel,
    mesh=_mesh,
    compiler_params=pltpu.CompilerParams(
        use_tc_tiling_on_sc=False, needs_layout_passes=False,
        disable_bounds_checks=True),
    out_type=jax.ShapeDtypeStruct((T, D // 8, NW, 8 * BW), jnp.float32),
    scratch_types=[
        pltpu.VMEM((T, BW), jnp.int32),
        pltpu.VMEM((2, BW, D), jnp.float32),
        pltpu.VMEM((2, D * BW), jnp.float32),
        pltpu.SemaphoreType.DMA((2,)),
        pltpu.SemaphoreType.DMA((2,)),
    ],
)
def _emb_lookup(w_hbm, xt_hbm, out_hbm, idx_v, raw_v, trans_v, gsem, wsem):
    wid = lax.axis_index("s") * NC + lax.axis_index("c")
    b0 = wid * BW
    pltpu.sync_copy(xt_hbm.at[:, pl.ds(b0, BW)], idx_v)

    def g_copy(t, j):
        return pltpu.make_async_copy(
            w_hbm.at[idx_v.at[t]], raw_v.at[j], gsem.at[j])

    def wb_copies(t, j):
        return [
            pltpu.make_async_copy(
                trans_v.at[j, pl.ds(dt * 8 * BW, 8 * BW)],
                out_hbm.at[t, dt, wid],
                wsem.at[j])
            for dt in range(D // 8)
        ]

    iota16 = lax.iota(jnp.int32, 16)
    row_ids = [iota16 + i0 * 16 for i0 in range(BW // 16)]

    def transpose(j):
        @plsc.parallel_loop(0, D, unroll=8)
        def per_d(d):
            colv = (iota16 + d) & (D - 1)
            colv128 = colv << 7
            for i0 in range(BW // 16):
                v = plsc.load_gather(raw_v.at[j], [row_ids[i0], colv])
                plsc.store_scatter(trans_v.at[j], [colv128 + row_ids[i0]], v)

    g_copy(0, 0).start()
    g_copy(1, 1).start()

    def pair(p, carry):
        t0 = p * 2
        for j in (0, 1):
            t = t0 + j
            g_copy(t, j).wait()

            @pl.when(t >= 2)
            def _():
                for cp in wb_copies(t - 2, j):
                    cp.wait()

            transpose(j)
            for cp in wb_copies(t, j):
                cp.start()

            @pl.when(t + 2 < T)
            def _():
                g_copy(t + 2, j).start()

        return carry

    lax.fori_loop(0, T // 2, pair, 0)
    for cp in wb_copies(T - 2, 0):
        cp.wait()
    for cp in wb_copies(T - 1, 1):
        cp.wait()


def kernel(X, W):
    x_t = jnp.transpose(X).astype(jnp.int32)
    out5 = _emb_lookup(W, x_t).reshape(T, D // 8, NW, 8, BW)
    return jnp.transpose(out5, (2, 4, 0, 1, 3)).reshape(NB, T, D)


# speedup vs baseline: 5.0052x; 1.0037x over previous
import functools

import jax
import jax.numpy as jnp
from jax import lax
from jax.experimental import pallas as pl
from jax.experimental.pallas import tpu as pltpu
from jax.experimental.pallas import tpu_sc as plsc

D = 64
NB = 4096
T = 200
BW = 128

_info = plsc.get_sparse_core_info()
NC, NS = _info.num_cores, _info.num_subcores
NW = NC * NS

_mesh = plsc.VectorSubcoreMesh(core_axis_name="c", subcore_axis_name="s")


@functools.partial(
    pl.kernel,
    mesh=_mesh,
    compiler_params=pltpu.CompilerParams(
        use_tc_tiling_on_sc=False, needs_layout_passes=False,
        disable_bounds_checks=True),
    out_type=jax.ShapeDtypeStruct((T, D // 8, NW, 8 * BW), jnp.float32),
    scratch_types=[
        pltpu.VMEM((T, BW), jnp.int32),
        pltpu.VMEM((2, BW, D), jnp.float32),
        pltpu.VMEM((2, D * BW), jnp.float32),
        pltpu.SemaphoreType.DMA((2,)),
        pltpu.SemaphoreType.DMA((2,)),
    ],
)
def _emb_lookup(w_hbm, xt_hbm, out_hbm, idx_v, raw_v, trans_v, gsem, wsem):
    wid = lax.axis_index("s") * NC + lax.axis_index("c")
    b0 = wid * BW
    pltpu.sync_copy(xt_hbm.at[:, pl.ds(b0, BW)], idx_v)

    def g_copy(t, j):
        return pltpu.make_async_copy(
            w_hbm.at[idx_v.at[t]], raw_v.at[j], gsem.at[j])

    def wb_copies(t, j):
        return [
            pltpu.make_async_copy(
                trans_v.at[j, pl.ds(dt * 8 * BW, 8 * BW)],
                out_hbm.at[t, dt, wid],
                wsem.at[j])
            for dt in range(D // 8)
        ]

    iota16 = lax.iota(jnp.int32, 16)
    row_ids = [iota16 + i0 * 16 for i0 in range(BW // 16)]

    def transpose(j):
        @plsc.parallel_loop(0, D, unroll=16)
        def per_d(d):
            colv = (iota16 + d) & (D - 1)
            colv128 = colv << 7
            for i0 in range(BW // 16):
                v = plsc.load_gather(raw_v.at[j], [row_ids[i0], colv])
                plsc.store_scatter(trans_v.at[j], [colv128 + row_ids[i0]], v)

    g_copy(0, 0).start()
    g_copy(1, 1).start()

    def pair(p, carry):
        t0 = p * 2
        for j in (0, 1):
            t = t0 + j
            g_copy(t, j).wait()

            @pl.when(t >= 2)
            def _():
                for cp in wb_copies(t - 2, j):
                    cp.wait()

            transpose(j)
            for cp in wb_copies(t, j):
                cp.start()

            @pl.when(t + 2 < T)
            def _():
                g_copy(t + 2, j).start()

        return carry

    lax.fori_loop(0, T // 2, pair, 0)
    for cp in wb_copies(T - 2, 0):
        cp.wait()
    for cp in wb_copies(T - 1, 1):
        cp.wait()


def kernel(X, W):
    x_t = jnp.transpose(X).astype(jnp.int32)
    out5 = _emb_lookup(W, x_t).reshape(T, D // 8, NW, 8, BW)
    return jnp.transpose(out5, (2, 4, 0, 1, 3)).reshape(NB, T, D)


# speedup vs baseline: 5.8288x; 1.1645x over previous
import functools

import jax
import jax.numpy as jnp
from jax import lax
from jax.experimental import pallas as pl
from jax.experimental.pallas import tpu as pltpu
from jax.experimental.pallas import tpu_sc as plsc

D = 64
NB = 4096
T = 200
BW = 128

_info = plsc.get_sparse_core_info()
NC, NS = _info.num_cores, _info.num_subcores
NW = NC * NS

_mesh = plsc.VectorSubcoreMesh(core_axis_name="c", subcore_axis_name="s")


@functools.partial(
    pl.kernel,
    mesh=_mesh,
    compiler_params=pltpu.CompilerParams(
        use_tc_tiling_on_sc=False, needs_layout_passes=False,
        disable_bounds_checks=True),
    out_type=jax.ShapeDtypeStruct((T, D // 8, NW, 8 * BW), jnp.float32),
    scratch_types=[
        pltpu.VMEM((T, BW), jnp.int32),
        pltpu.VMEM((4, BW, D), jnp.float32),
        pltpu.VMEM((4, D * BW), jnp.float32),
        pltpu.SemaphoreType.DMA((4,)),
        pltpu.SemaphoreType.DMA((4,)),
    ],
)
def _emb_lookup(w_hbm, xt_hbm, out_hbm, idx_v, raw_v, trans_v, gsem, wsem):
    wid = lax.axis_index("s") * NC + lax.axis_index("c")
    b0 = wid * BW
    pltpu.sync_copy(xt_hbm.at[:, pl.ds(b0, BW)], idx_v)

    def g_copy(t, j):
        return pltpu.make_async_copy(
            w_hbm.at[idx_v.at[t]], raw_v.at[j], gsem.at[j])

    def wb_copies(t, j):
        return [
            pltpu.make_async_copy(
                trans_v.at[j, pl.ds(dt * 8 * BW, 8 * BW)],
                out_hbm.at[t, dt, wid],
                wsem.at[j])
            for dt in range(D // 8)
        ]

    iota16 = lax.iota(jnp.int32, 16)
    row_ids = [iota16 + i0 * 16 for i0 in range(BW // 16)]

    def transpose(j):
        @plsc.parallel_loop(0, D, unroll=16)
        def per_d(d):
            colv = (iota16 + d) & (D - 1)
            colv128 = colv << 7
            for i0 in range(BW // 16):
                v = plsc.load_gather(raw_v.at[j], [row_ids[i0], colv])
                plsc.store_scatter(trans_v.at[j], [colv128 + row_ids[i0]], v)

    NBUF = 4
    for j in range(NBUF):
        g_copy(j, j).start()

    def quad(p, carry):
        t0 = p * NBUF
        for j in range(NBUF):
            t = t0 + j
            g_copy(t, j).wait()

            @pl.when(t >= NBUF)
            def _():
                for cp in wb_copies(t - NBUF, j):
                    cp.wait()

            transpose(j)
            for cp in wb_copies(t, j):
                cp.start()

            @pl.when(t + NBUF < T)
            def _():
                g_copy(t + NBUF, j).start()

        return carry

    lax.fori_loop(0, T // NBUF, quad, 0)
    for j in range(NBUF):
        for cp in wb_copies(T - NBUF + j, j):
            cp.wait()


def kernel(X, W):
    x_t = jnp.transpose(X).astype(jnp.int32)
    out5 = _emb_lookup(W, x_t).reshape(T, D // 8, NW, 8, BW)
    return jnp.transpose(out5, (2, 4, 0, 1, 3)).reshape(NB, T, D)
